# baseline (speedup 1.0000x reference)
import functools

import jax
import jax.numpy as jnp
from jax import lax
from jax.experimental import pallas as pl
from jax.experimental.pallas import tpu as pltpu
from jax.experimental.pallas import tpu_sc as plsc

_I = False

NPAD = 10240
T = 20
D = 128
DEG = 16
TOPK = 8
CORE_EVERY = 100
BLK = 512
RBLK = 256
IBLK = 2000
NITEM_BLOCKS = 50


def _dot(a, b):
    return jnp.dot(a, b, preferred_element_type=jnp.float32)


NC = 2
NS = 16
NW = NC * NS

_MESH = dict(core_axis_name="c", subcore_axis_name="s")


def _chunks(sub):
    out, o = [], 0
    while o < sub:
        ch = min(128, sub - o)
        out.append((o, ch))
        o += ch
    return out


def _sc_gather(table, idx, sub, u01):
    total = idx.shape[0]
    u0, u1 = u01
    assert NS * (u0 + u1) * sub == total and u0 % 2 == 0 and u1 % 2 == 0
    ck = _chunks(sub)

    @functools.partial(
        pl.kernel, mesh=plsc.VectorSubcoreMesh(**_MESH),
        out_type=jax.ShapeDtypeStruct((total, D), jnp.float32),
        scratch_types=[pltpu.VMEM((sub,), jnp.int32),
                       pltpu.VMEM((sub,), jnp.int32),
                       pltpu.VMEM((sub, D), jnp.float32),
                       pltpu.VMEM((sub, D), jnp.float32),
                       pltpu.SemaphoreType.DMA,
                       pltpu.SemaphoreType.DMA,
                       pltpu.SemaphoreType.DMA])
    def k(table_hbm, idx_hbm, out_hbm, idx0, idx1, rows0, rows1,
          semg, semo0, semo1):
        c = lax.axis_index("c")
        s = lax.axis_index("s")
        base = jnp.where(c == 0, s * u0 * sub,
                         (NS * u0 + s * u1) * sub)
        nsub = jnp.where(c == 0, u0, u1)
        idxb = (idx0, idx1)
        rowsb = (rows0, rows1)
        semo = (semo0, semo1)

        def body(kk, carry):
            hs = [[], []]
            for b in (0, 1):
                off = base + (2 * kk + b) * sub
                @pl.when(kk > 0)
                def _wait():
                    pltpu.make_async_copy(out_hbm.at[pl.ds(0, sub)],
                                          rowsb[b], semo[b]).wait()
                pltpu.sync_copy(idx_hbm.at[pl.ds(off, sub)], idxb[b])
                for (o, ch) in ck:
                    hs[b].append(pltpu.async_copy(
                        table_hbm.at[idxb[b].at[pl.ds(o, ch)]],
                        rowsb[b].at[pl.ds(o, ch)], semg))
            for b in (0, 1):
                off = base + (2 * kk + b) * sub
                for h in hs[b]:
                    h.wait()
                pltpu.async_copy(rowsb[b], out_hbm.at[pl.ds(off, sub)], semo[b])
            return carry

        lax.fori_loop(0, nsub // 2, body, 0)
        for b in (0, 1):
            pltpu.make_async_copy(out_hbm.at[pl.ds(0, sub)], rowsb[b],
                                  semo[b]).wait()

    return k(table, idx)


def _sc_gather_edges(ur, buf, nid, idx_perm, idx_row):
    total = idx_perm.shape[0]
    sub = 160
    u0, u1 = 42, 22
    assert NS * (u0 + u1) * sub == total
    ck = _chunks(sub)

    @functools.partial(
        pl.kernel, mesh=plsc.VectorSubcoreMesh(**_MESH),
        out_type=(jax.ShapeDtypeStruct((total, D), jnp.float32),
                  jax.ShapeDtypeStruct((total, D), jnp.float32),
                  jax.ShapeDtypeStruct((total,), jnp.int32)),
        scratch_types=[pltpu.VMEM((sub,), jnp.int32),
                       pltpu.VMEM((sub,), jnp.int32),
                       pltpu.VMEM((sub,), jnp.int32),
                       pltpu.VMEM((sub,), jnp.int32),
                       pltpu.VMEM((sub, D), jnp.float32),
                       pltpu.VMEM((sub, D), jnp.float32),
                       pltpu.VMEM((sub, D), jnp.float32),
                       pltpu.VMEM((sub, D), jnp.float32),
                       pltpu.VMEM((sub,), jnp.int32),
                       pltpu.VMEM((sub,), jnp.int32),
                       pltpu.SemaphoreType.DMA,
                       pltpu.SemaphoreType.DMA,
                       pltpu.SemaphoreType.DMA])
    def k(ur_hbm, buf_hbm, nid_hbm, idxp_hbm, idxr_hbm,
          srg_hbm, bg_hbm, sid_hbm,
          idxp0, idxp1, idxr0, idxr1, ra0, ra1, rb0, rb1, rs0, rs1,
          semg, semo0, semo1):
        c = lax.axis_index("c")
        s = lax.axis_index("s")
        base = jnp.where(c == 0, s * u0 * sub,
                         (NS * u0 + s * u1) * sub)
        nsub = jnp.where(c == 0, u0, u1)
        idxp = (idxp0, idxp1)
        idxr = (idxr0, idxr1)
        ra = (ra0, ra1)
        rb = (rb0, rb1)
        rs = (rs0, rs1)
        semo = (semo0, semo1)

        def body(kk, carry):
            hs = [[], []]
            for b in (0, 1):
                off = base + (2 * kk + b) * sub

                @pl.when(kk > 0)
                def _wait():
                    pltpu.make_async_copy(srg_hbm.at[pl.ds(0, sub)],
                                          ra[b], semo[b]).wait()
                    pltpu.make_async_copy(bg_hbm.at[pl.ds(0, sub)],
                                          rb[b], semo[b]).wait()
                    pltpu.make_async_copy(sid_hbm.at[pl.ds(0, sub)],
                                          rs[b], semo[b]).wait()
                pltpu.sync_copy(idxp_hbm.at[pl.ds(off, sub)], idxp[b])
                pltpu.sync_copy(idxr_hbm.at[pl.ds(off, sub)], idxr[b])
                for (o, ch) in ck:
                    ip = idxp[b].at[pl.ds(o, ch)]
                    ir = idxr[b].at[pl.ds(o, ch)]
                    hs[b].append(pltpu.async_copy(ur_hbm.at[ip],
                                                  ra[b].at[pl.ds(o, ch)], semg))
                    hs[b].append(pltpu.async_copy(buf_hbm.at[ip],
                                                  rb[b].at[pl.ds(o, ch)], semg))
                    hs[b].append(pltpu.async_copy(nid_hbm.at[ir],
                                                  rs[b].at[pl.ds(o, ch)], semg))
            for b in (0, 1):
                off = base + (2 * kk + b) * sub
                for h in hs[b]:
                    h.wait()
                pltpu.async_copy(ra[b], srg_hbm.at[pl.ds(off, sub)], semo[b])
                pltpu.async_copy(rb[b], bg_hbm.at[pl.ds(off, sub)], semo[b])
                pltpu.async_copy(rs[b], sid_hbm.at[pl.ds(off, sub)], semo[b])
            return carry

        lax.fori_loop(0, nsub // 2, body, 0)
        for b in (0, 1):
            pltpu.make_async_copy(srg_hbm.at[pl.ds(0, sub)], ra[b],
                                  semo[b]).wait()
            pltpu.make_async_copy(bg_hbm.at[pl.ds(0, sub)], rb[b],
                                  semo[b]).wait()
            pltpu.make_async_copy(sid_hbm.at[pl.ds(0, sub)], rs[b],
                                  semo[b]).wait()

    return k(ur, buf, nid, idx_perm, idx_row)


def _lstm_body(x_ref, len_ref, wih_ref, whh_ref, bih_ref, bhh_ref, out_ref):
    x = x_ref[...]
    wih = wih_ref[...]
    whh = whh_ref[...]
    bih = bih_ref[...]
    bhh = bhh_ref[...]
    ln = len_ref[...]
    bsz = x.shape[0]
    h = jnp.zeros((bsz, D), jnp.float32)
    c = jnp.zeros((bsz, D), jnp.float32)
    ho = jnp.zeros((bsz, D), jnp.float32)
    for t in range(T):
        xt = x[:, D * t:D * (t + 1)]
        g = (_dot(xt, wih) + bih + _dot(h, whh)) + bhh
        i = jax.nn.sigmoid(g[:, :D])
        f = jax.nn.sigmoid(g[:, D:2 * D])
        gg = jnp.tanh(g[:, 2 * D:3 * D])
        o = jax.nn.sigmoid(g[:, 3 * D:])
        c = f * c + i * gg
        h = o * jnp.tanh(c)
        ho = jnp.where(ln == (t + 1), h, ho)
    out_ref[...] = ho


def _lstm(xg, len2, wih_t, whh_t, bih, bhh):
    return pl.pallas_call(
        _lstm_body,
        grid=(NPAD // BLK,),
        in_specs=[
            pl.BlockSpec((BLK, T * D), lambda i: (i, 0)),
            pl.BlockSpec((BLK, 1), lambda i: (i, 0)),
            pl.BlockSpec((D, 4 * D), lambda i: (0, 0)),
            pl.BlockSpec((D, 4 * D), lambda i: (0, 0)),
            pl.BlockSpec((1, 4 * D), lambda i: (0, 0)),
            pl.BlockSpec((1, 4 * D), lambda i: (0, 0)),
        ],
        out_specs=pl.BlockSpec((BLK, D), lambda i: (i, 0)),
        out_shape=jax.ShapeDtypeStruct((NPAD, D), jnp.float32),
        interpret=_I,
    )(xg, len2, wih_t, whh_t, bih, bhh)


def _fuse_body(ur_ref, ug_ref, core_ref, w1t_ref, buf_ref):
    ur = ur_ref[...]
    feat = jnp.concatenate([ug_ref[...], ur], axis=1)
    fr = jnp.maximum(_dot(feat, w1t_ref[...]), 0.0)
    buf_ref[...] = jnp.where(core_ref[...] == 1, ur, fr)


def _fuse(ur, ug, core2, w1t):
    row = lambda w: pl.BlockSpec((BLK, w), lambda i: (i, 0))
    return pl.pallas_call(
        _fuse_body,
        grid=(NPAD // BLK,),
        in_specs=[row(D), row(D), row(1),
                  pl.BlockSpec((2 * D, D), lambda i: (0, 0))],
        out_specs=row(D),
        out_shape=jax.ShapeDtypeStruct((NPAD, D), jnp.float32),
        interpret=_I,
    )(ur, ug, core2, w1t)


def _r1_body(srg_ref, bg_ref, sid_ref, ur_ref, buf_ref, init_ref, kr_ref,
             mw1t_ref, mb1_ref, mw2_ref, agg_ref, ps_ref, sel_ref):
    bsz = ur_ref.shape[0]
    ur = ur_ref[...]
    mw1t = mw1t_ref[...]
    mb1 = mb1_ref[...]
    lane = lax.broadcasted_iota(jnp.int32, (bsz, DEG), 1)
    s = jnp.zeros((bsz, DEG), jnp.float32)
    w2row = mw2_ref[...].astype(jnp.bfloat16).astype(jnp.float32)
    for j in range(DEG):
        sr = srg_ref[j]
        feat = jnp.concatenate([sr, sr - ur], axis=1)
        hid = jnp.maximum(_dot(feat, mw1t) + mb1, 0.0)
        hb = hid.astype(jnp.bfloat16).astype(jnp.float32)
        sj = jnp.sum(hb * w2row, axis=1, keepdims=True)
        s = s + jnp.where(lane == j, sj, 0.0)
    lane8 = lax.broadcasted_iota(jnp.int32, (bsz, TOPK), 1)
    work = s
    mask = jnp.zeros((bsz, DEG), jnp.float32)
    selv = jnp.zeros((bsz, TOPK), jnp.int32)
    sid = sid_ref[...]
    for k in range(TOPK):
        m = jnp.max(work, axis=1, keepdims=True)
        am = jnp.min(jnp.where(work == m, lane, DEG), axis=1, keepdims=True)
        oh = lane == am
        mask = mask + jnp.where(oh, 1.0, 0.0)
        selk = jnp.sum(jnp.where(oh, sid, 0), axis=1, keepdims=True)
        selv = selv + jnp.where(lane8 == k, selk, 0)
        work = jnp.where(oh, -3e38, work)
    sel_ref[...] = selv
    kr = kr_ref[0, 0]
    ps = (1.0 - kr) * jax.nn.sigmoid(s - (1.0 - mask) * 1e11) \
        + kr * init_ref[...]
    ps_ref[...] = ps
    coeff = jnp.sum(ps, axis=1, keepdims=True) + 1.0
    fr = jnp.zeros((bsz, D), jnp.float32)
    for j in range(DEG):
        fr = fr + bg_ref[j] * ps[:, j:j + 1]
    agg_ref[...] = (fr + buf_ref[...]) / coeff


def _round1(srg, bg, sid, ur, buf, initm, kr2, mw1t, mb1r, mw2r):
    edge3 = pl.BlockSpec((DEG, RBLK, D), lambda i: (0, i, 0))
    row = lambda w: pl.BlockSpec((RBLK, w), lambda i: (i, 0))
    one = lambda a, b: pl.BlockSpec((a, b), lambda i: (0, 0))
    return pl.pallas_call(
        _r1_body,
        grid=(NPAD // RBLK,),
        in_specs=[edge3, edge3, row(DEG), row(D), row(D), row(DEG),
                  one(1, 1), one(2 * D, D), one(1, D), one(1, D)],
        out_specs=[row(D), row(DEG), row(TOPK)],
        out_shape=[jax.ShapeDtypeStruct((NPAD, D), jnp.float32),
                   jax.ShapeDtypeStruct((NPAD, DEG), jnp.float32),
                   jax.ShapeDtypeStruct((NPAD, TOPK), jnp.int32)],
        interpret=_I,
    )(srg, bg, sid, ur, buf, initm, kr2, mw1t, mb1r, mw2r)


def _r2_body(ag_ref, agg1_ref, ps_ref, out_ref):
    bsz = agg1_ref.shape[0]
    ps = ps_ref[...]
    coeff = jnp.sum(ps, axis=1, keepdims=True) + 1.0
    fr = jnp.zeros((bsz, D), jnp.float32)
    for j in range(DEG):
        fr = fr + ag_ref[j] * ps[:, j:j + 1]
    out_ref[...] = (fr + agg1_ref[...]) / coeff


def _round2(ag, agg1, ps):
    edge3 = pl.BlockSpec((DEG, RBLK, D), lambda i: (0, i, 0))
    row = lambda w: pl.BlockSpec((RBLK, w), lambda i: (i, 0))
    return pl.pallas_call(
        _r2_body,
        grid=(NPAD // RBLK,),
        in_specs=[edge3, row(D), row(DEG)],
        out_specs=row(D),
        out_shape=jax.ShapeDtypeStruct((NPAD, D), jnp.float32),
        interpret=_I,
    )(ag, agg1, ps)


def _sm1_body(ra_ref, rb_ref, ie_ref, bm_ref, bs_ref):
    rep = ra_ref[...] + rb_ref[...]
    logits = lax.dot_general(ie_ref[...], rep, (((1,), (1,)), ((), ())),
                             preferred_element_type=jnp.float32)
    m = jnp.max(logits, axis=0, keepdims=True)
    bm_ref[...] = m[None]
    bs_ref[...] = jnp.sum(jnp.exp(logits - m), axis=0, keepdims=True)[None]


def _sm2_body(ra_ref, rb_ref, ie_ref, bm_ref, bs_ref, out_ref):
    rep = ra_ref[...] + rb_ref[...]
    logits = lax.dot_general(ie_ref[...], rep, (((1,), (1,)), ((), ())),
                             preferred_element_type=jnp.float32)
    bm = bm_ref[...]
    m = jnp.max(bm, axis=0, keepdims=True)
    ssum = jnp.sum(bs_ref[...] * jnp.exp(bm - m), axis=0, keepdims=True)
    out_ref[...] = jnp.exp(logits - m) / ssum


def _softmax_scores(ra, rb, item_emb):
    rrow = pl.BlockSpec((D, D), lambda i: (0, 0))
    iblk = pl.BlockSpec((IBLK, D), lambda i: (i, 0))
    col = pl.BlockSpec((1, 1, D), lambda i: (i, 0, 0))
    stat = pl.BlockSpec((NITEM_BLOCKS, D), lambda i: (0, 0))
    bm, bs = pl.pallas_call(
        _sm1_body,
        grid=(NITEM_BLOCKS,),
        in_specs=[rrow, rrow, iblk],
        out_specs=[col, col],
        out_shape=[jax.ShapeDtypeStruct((NITEM_BLOCKS, 1, D), jnp.float32)] * 2,
        interpret=_I,
    )(ra, rb, item_emb)
    bm = bm.reshape(NITEM_BLOCKS, D)
    bs = bs.reshape(NITEM_BLOCKS, D)
    return pl.pallas_call(
        _sm2_body,
        grid=(NITEM_BLOCKS,),
        in_specs=[rrow, rrow, iblk, stat, stat],
        out_specs=pl.BlockSpec((IBLK, D), lambda i: (i, 0)),
        out_shape=jax.ShapeDtypeStruct((IBLK * NITEM_BLOCKS, D), jnp.float32),
        interpret=_I,
    )(ra, rb, item_emb, bm, bs)


def kernel(session, length, is_core, node_id, edge_index, initial_score,
           keep_rate, item_emb, user_emb, lw_ih, lw_hh, lb_ih, lb_hh,
           mw1, mb1, mw2, W1_w, W2_w, fc_w0, fc_w1):
    n = session.shape[0]
    pad = NPAD - n
    src = edge_index[0]

    sess_p = jnp.pad(session, ((0, pad), (0, 0)))
    len_p = jnp.pad(length, (0, pad), constant_values=1)
    len2 = len_p.reshape(NPAD, 1).astype(jnp.int32)
    nid_p = jnp.pad(node_id, (0, pad))
    core2 = jnp.pad(is_core, (0, pad)).reshape(NPAD, 1).astype(jnp.int32)
    srcm_p = jnp.pad(src.reshape(n, DEG), ((0, pad), (0, 0)))
    init_p = jnp.pad(initial_score.reshape(n, DEG), ((0, pad), (0, 0)))

    sess_flat = sess_p.reshape(-1).astype(jnp.int32)
    nid_p = nid_p.astype(jnp.int32)
    xg = _sc_gather(item_emb, sess_flat, 320, (32, 8)).reshape(NPAD, T * D)
    ug = _sc_gather(user_emb, nid_p, 160, (2, 2))

    ur = _lstm(xg, len2, lw_ih.T, lw_hh.T, lb_ih.reshape(1, 4 * D),
               lb_hh.reshape(1, 4 * D))

    buf = _fuse(ur, ug, core2, W1_w.T)

    idx_perm = srcm_p.T.reshape(-1).astype(jnp.int32)
    idx_row = srcm_p.reshape(-1).astype(jnp.int32)
    srg_f, bg_f, sid_f = _sc_gather_edges(ur, buf, nid_p, idx_perm, idx_row)
    srg = srg_f.reshape(DEG, NPAD, D)
    bg = bg_f.reshape(DEG, NPAD, D)
    sid = sid_f.reshape(NPAD, DEG)

    kr2 = jnp.asarray(keep_rate, jnp.float32).reshape(1, 1)
    agg1, ps, selp = _round1(srg, bg, sid, ur, buf, init_p, kr2,
                             mw1.T, mb1.reshape(1, D), mw2)

    ag = _sc_gather(agg1, idx_perm, 320, (24, 8)).reshape(DEG, NPAD, D)
    agg2 = _round2(ag, agg1, ps)

    recent = ur[0:n:CORE_EVERY]
    social = agg2[0:n:CORE_EVERY]
    ncore = n // CORE_EVERY
    rpad = ((0, D - ncore), (0, 0))
    ra = jnp.pad(recent, rpad)
    rb = jnp.pad(social, rpad)
    scorep = _softmax_scores(ra, rb, item_emb)

    score = scorep[:, :ncore].T
    sel = selp[0:n:CORE_EVERY]
    core_user = node_id[0:n:CORE_EVERY]
    return score, sel, core_user


# speedup vs baseline: 1.0080x; 1.0080x over previous
import functools

import jax
import jax.numpy as jnp
from jax import lax
from jax.experimental import pallas as pl
from jax.experimental.pallas import tpu as pltpu
from jax.experimental.pallas import tpu_sc as plsc

_I = False

NPAD = 10240
T = 20
D = 128
DEG = 16
TOPK = 8
CORE_EVERY = 100
BLK = 512
RBLK = 256
IBLK = 2000
NITEM_BLOCKS = 50


def _dot(a, b):
    return jnp.dot(a, b, preferred_element_type=jnp.float32)


NC = 2
NS = 16
NW = NC * NS

_MESH = dict(core_axis_name="c", subcore_axis_name="s")


def _chunks(sub):
    out, o = [], 0
    while o < sub:
        ch = min(128, sub - o)
        out.append((o, ch))
        o += ch
    return out


def _sc_gather(table, idx, sub, u01):
    total = idx.shape[0]
    u0, u1 = u01
    assert NS * (u0 + u1) * sub == total and u0 % 2 == 0 and u1 % 2 == 0
    ck = _chunks(sub)

    @functools.partial(
        pl.kernel, mesh=plsc.VectorSubcoreMesh(**_MESH),
        out_type=jax.ShapeDtypeStruct((total, D), jnp.float32),
        scratch_types=[pltpu.VMEM((sub,), jnp.int32),
                       pltpu.VMEM((sub,), jnp.int32),
                       pltpu.VMEM((sub, D), jnp.float32),
                       pltpu.VMEM((sub, D), jnp.float32),
                       pltpu.SemaphoreType.DMA,
                       pltpu.SemaphoreType.DMA,
                       pltpu.SemaphoreType.DMA])
    def k(table_hbm, idx_hbm, out_hbm, idx0, idx1, rows0, rows1,
          semg, semo0, semo1):
        wid = lax.axis_index("s") * NC + lax.axis_index("c")
        nsub = (u0 + u1) // 2
        base = wid * nsub * sub
        idxb = (idx0, idx1)
        rowsb = (rows0, rows1)
        semo = (semo0, semo1)

        def body(kk, carry):
            hs = [[], []]
            for b in (0, 1):
                off = base + (2 * kk + b) * sub
                @pl.when(kk > 0)
                def _wait():
                    pltpu.make_async_copy(out_hbm.at[pl.ds(0, sub)],
                                          rowsb[b], semo[b]).wait()
                pltpu.sync_copy(idx_hbm.at[pl.ds(off, sub)], idxb[b])
                for (o, ch) in ck:
                    hs[b].append(pltpu.async_copy(
                        table_hbm.at[idxb[b].at[pl.ds(o, ch)]],
                        rowsb[b].at[pl.ds(o, ch)], semg))
            for b in (0, 1):
                off = base + (2 * kk + b) * sub
                for h in hs[b]:
                    h.wait()
                pltpu.async_copy(rowsb[b], out_hbm.at[pl.ds(off, sub)], semo[b])
            return carry

        lax.fori_loop(0, nsub // 2, body, 0)
        for b in (0, 1):
            pltpu.make_async_copy(out_hbm.at[pl.ds(0, sub)], rowsb[b],
                                  semo[b]).wait()

    return k(table, idx)


def _sc_gather_edges(ur, buf, nid, idx_perm, idx_row):
    total = idx_perm.shape[0]
    sub = 160
    u0, u1 = 42, 22
    assert NS * (u0 + u1) * sub == total
    ck = _chunks(sub)

    @functools.partial(
        pl.kernel, mesh=plsc.VectorSubcoreMesh(**_MESH),
        out_type=(jax.ShapeDtypeStruct((total, D), jnp.float32),
                  jax.ShapeDtypeStruct((total, D), jnp.float32),
                  jax.ShapeDtypeStruct((total,), jnp.int32)),
        scratch_types=[pltpu.VMEM((sub,), jnp.int32),
                       pltpu.VMEM((sub,), jnp.int32),
                       pltpu.VMEM((sub,), jnp.int32),
                       pltpu.VMEM((sub,), jnp.int32),
                       pltpu.VMEM((sub, D), jnp.float32),
                       pltpu.VMEM((sub, D), jnp.float32),
                       pltpu.VMEM((sub, D), jnp.float32),
                       pltpu.VMEM((sub, D), jnp.float32),
                       pltpu.VMEM((sub,), jnp.int32),
                       pltpu.VMEM((sub,), jnp.int32),
                       pltpu.SemaphoreType.DMA,
                       pltpu.SemaphoreType.DMA,
                       pltpu.SemaphoreType.DMA])
    def k(ur_hbm, buf_hbm, nid_hbm, idxp_hbm, idxr_hbm,
          srg_hbm, bg_hbm, sid_hbm,
          idxp0, idxp1, idxr0, idxr1, ra0, ra1, rb0, rb1, rs0, rs1,
          semg, semo0, semo1):
        wid = lax.axis_index("s") * NC + lax.axis_index("c")
        nsub = (u0 + u1) // 2
        base = wid * nsub * sub
        idxp = (idxp0, idxp1)
        idxr = (idxr0, idxr1)
        ra = (ra0, ra1)
        rb = (rb0, rb1)
        rs = (rs0, rs1)
        semo = (semo0, semo1)

        def body(kk, carry):
            hs = [[], []]
            for b in (0, 1):
                off = base + (2 * kk + b) * sub

                @pl.when(kk > 0)
                def _wait():
                    pltpu.make_async_copy(srg_hbm.at[pl.ds(0, sub)],
                                          ra[b], semo[b]).wait()
                    pltpu.make_async_copy(bg_hbm.at[pl.ds(0, sub)],
                                          rb[b], semo[b]).wait()
                    pltpu.make_async_copy(sid_hbm.at[pl.ds(0, sub)],
                                          rs[b], semo[b]).wait()
                pltpu.sync_copy(idxp_hbm.at[pl.ds(off, sub)], idxp[b])
                pltpu.sync_copy(idxr_hbm.at[pl.ds(off, sub)], idxr[b])
                for (o, ch) in ck:
                    ip = idxp[b].at[pl.ds(o, ch)]
                    ir = idxr[b].at[pl.ds(o, ch)]
                    hs[b].append(pltpu.async_copy(ur_hbm.at[ip],
                                                  ra[b].at[pl.ds(o, ch)], semg))
                    hs[b].append(pltpu.async_copy(buf_hbm.at[ip],
                                                  rb[b].at[pl.ds(o, ch)], semg))
                    hs[b].append(pltpu.async_copy(nid_hbm.at[ir],
                                                  rs[b].at[pl.ds(o, ch)], semg))
            for b in (0, 1):
                off = base + (2 * kk + b) * sub
                for h in hs[b]:
                    h.wait()
                pltpu.async_copy(ra[b], srg_hbm.at[pl.ds(off, sub)], semo[b])
                pltpu.async_copy(rb[b], bg_hbm.at[pl.ds(off, sub)], semo[b])
                pltpu.async_copy(rs[b], sid_hbm.at[pl.ds(off, sub)], semo[b])
            return carry

        lax.fori_loop(0, nsub // 2, body, 0)
        for b in (0, 1):
            pltpu.make_async_copy(srg_hbm.at[pl.ds(0, sub)], ra[b],
                                  semo[b]).wait()
            pltpu.make_async_copy(bg_hbm.at[pl.ds(0, sub)], rb[b],
                                  semo[b]).wait()
            pltpu.make_async_copy(sid_hbm.at[pl.ds(0, sub)], rs[b],
                                  semo[b]).wait()

    return k(ur, buf, nid, idx_perm, idx_row)


def _lstm_body(x_ref, len_ref, wih_ref, whh_ref, bih_ref, bhh_ref, out_ref):
    x = x_ref[...]
    wih = wih_ref[...]
    whh = whh_ref[...]
    bih = bih_ref[...]
    bhh = bhh_ref[...]
    ln = len_ref[...]
    bsz = x.shape[0]
    h = jnp.zeros((bsz, D), jnp.float32)
    c = jnp.zeros((bsz, D), jnp.float32)
    ho = jnp.zeros((bsz, D), jnp.float32)
    for t in range(T):
        xt = x[:, D * t:D * (t + 1)]
        g = (_dot(xt, wih) + bih + _dot(h, whh)) + bhh
        i = jax.nn.sigmoid(g[:, :D])
        f = jax.nn.sigmoid(g[:, D:2 * D])
        gg = jnp.tanh(g[:, 2 * D:3 * D])
        o = jax.nn.sigmoid(g[:, 3 * D:])
        c = f * c + i * gg
        h = o * jnp.tanh(c)
        ho = jnp.where(ln == (t + 1), h, ho)
    out_ref[...] = ho


def _lstm(xg, len2, wih_t, whh_t, bih, bhh):
    return pl.pallas_call(
        _lstm_body,
        grid=(NPAD // BLK,),
        in_specs=[
            pl.BlockSpec((BLK, T * D), lambda i: (i, 0)),
            pl.BlockSpec((BLK, 1), lambda i: (i, 0)),
            pl.BlockSpec((D, 4 * D), lambda i: (0, 0)),
            pl.BlockSpec((D, 4 * D), lambda i: (0, 0)),
            pl.BlockSpec((1, 4 * D), lambda i: (0, 0)),
            pl.BlockSpec((1, 4 * D), lambda i: (0, 0)),
        ],
        out_specs=pl.BlockSpec((BLK, D), lambda i: (i, 0)),
        out_shape=jax.ShapeDtypeStruct((NPAD, D), jnp.float32),
        interpret=_I,
    )(xg, len2, wih_t, whh_t, bih, bhh)


def _fuse_body(ur_ref, ug_ref, core_ref, w1t_ref, buf_ref):
    ur = ur_ref[...]
    feat = jnp.concatenate([ug_ref[...], ur], axis=1)
    fr = jnp.maximum(_dot(feat, w1t_ref[...]), 0.0)
    buf_ref[...] = jnp.where(core_ref[...] == 1, ur, fr)


def _fuse(ur, ug, core2, w1t):
    row = lambda w: pl.BlockSpec((BLK, w), lambda i: (i, 0))
    return pl.pallas_call(
        _fuse_body,
        grid=(NPAD // BLK,),
        in_specs=[row(D), row(D), row(1),
                  pl.BlockSpec((2 * D, D), lambda i: (0, 0))],
        out_specs=row(D),
        out_shape=jax.ShapeDtypeStruct((NPAD, D), jnp.float32),
        interpret=_I,
    )(ur, ug, core2, w1t)


def _r1_body(srg_ref, bg_ref, sid_ref, ur_ref, buf_ref, init_ref, kr_ref,
             mw1t_ref, mb1_ref, mw2_ref, agg_ref, ps_ref, sel_ref):
    bsz = ur_ref.shape[0]
    ur = ur_ref[...]
    mw1t = mw1t_ref[...]
    mb1 = mb1_ref[...]
    lane = lax.broadcasted_iota(jnp.int32, (bsz, DEG), 1)
    s = jnp.zeros((bsz, DEG), jnp.float32)
    mw2t = mw2_ref[...]
    for j in range(DEG):
        sr = srg_ref[j]
        feat = jnp.concatenate([sr, sr - ur], axis=1)
        hid = jnp.maximum(_dot(feat, mw1t) + mb1, 0.0)
        sj = _dot(hid, mw2t)
        s = s + jnp.where(lane == j, sj, 0.0)
    lane8 = lax.broadcasted_iota(jnp.int32, (bsz, TOPK), 1)
    work = s
    mask = jnp.zeros((bsz, DEG), jnp.float32)
    selv = jnp.zeros((bsz, TOPK), jnp.int32)
    sid = sid_ref[...]
    for k in range(TOPK):
        m = jnp.max(work, axis=1, keepdims=True)
        am = jnp.min(jnp.where(work == m, lane, DEG), axis=1, keepdims=True)
        oh = lane == am
        mask = mask + jnp.where(oh, 1.0, 0.0)
        selk = jnp.sum(jnp.where(oh, sid, 0), axis=1, keepdims=True)
        selv = selv + jnp.where(lane8 == k, selk, 0)
        work = jnp.where(oh, -3e38, work)
    sel_ref[...] = selv
    kr = kr_ref[0, 0]
    ps = (1.0 - kr) * jax.nn.sigmoid(s - (1.0 - mask) * 1e11) \
        + kr * init_ref[...]
    ps_ref[...] = ps
    coeff = jnp.sum(ps, axis=1, keepdims=True) + 1.0
    fr = jnp.zeros((bsz, D), jnp.float32)
    for j in range(DEG):
        fr = fr + bg_ref[j] * ps[:, j:j + 1]
    agg_ref[...] = (fr + buf_ref[...]) / coeff


def _round1(srg, bg, sid, ur, buf, initm, kr2, mw1t, mb1r, mw2r):
    edge3 = pl.BlockSpec((DEG, RBLK, D), lambda i: (0, i, 0))
    row = lambda w: pl.BlockSpec((RBLK, w), lambda i: (i, 0))
    one = lambda a, b: pl.BlockSpec((a, b), lambda i: (0, 0))
    return pl.pallas_call(
        _r1_body,
        grid=(NPAD // RBLK,),
        in_specs=[edge3, edge3, row(DEG), row(D), row(D), row(DEG),
                  one(1, 1), one(2 * D, D), one(1, D), one(D, 1)],
        out_specs=[row(D), row(DEG), row(TOPK)],
        out_shape=[jax.ShapeDtypeStruct((NPAD, D), jnp.float32),
                   jax.ShapeDtypeStruct((NPAD, DEG), jnp.float32),
                   jax.ShapeDtypeStruct((NPAD, TOPK), jnp.int32)],
        interpret=_I,
    )(srg, bg, sid, ur, buf, initm, kr2, mw1t, mb1r, mw2r)


def _r2_body(ag_ref, agg1_ref, ps_ref, out_ref):
    bsz = agg1_ref.shape[0]
    ps = ps_ref[...]
    coeff = jnp.sum(ps, axis=1, keepdims=True) + 1.0
    fr = jnp.zeros((bsz, D), jnp.float32)
    for j in range(DEG):
        fr = fr + ag_ref[j] * ps[:, j:j + 1]
    out_ref[...] = (fr + agg1_ref[...]) / coeff


def _round2(ag, agg1, ps):
    edge3 = pl.BlockSpec((DEG, RBLK, D), lambda i: (0, i, 0))
    row = lambda w: pl.BlockSpec((RBLK, w), lambda i: (i, 0))
    return pl.pallas_call(
        _r2_body,
        grid=(NPAD // RBLK,),
        in_specs=[edge3, row(D), row(DEG)],
        out_specs=row(D),
        out_shape=jax.ShapeDtypeStruct((NPAD, D), jnp.float32),
        interpret=_I,
    )(ag, agg1, ps)


def _sm1_body(ra_ref, rb_ref, ie_ref, bm_ref, bs_ref):
    rep = ra_ref[...] + rb_ref[...]
    logits = lax.dot_general(ie_ref[...], rep, (((1,), (1,)), ((), ())),
                             preferred_element_type=jnp.float32)
    m = jnp.max(logits, axis=0, keepdims=True)
    bm_ref[...] = m[None]
    bs_ref[...] = jnp.sum(jnp.exp(logits - m), axis=0, keepdims=True)[None]


def _sm2_body(ra_ref, rb_ref, ie_ref, bm_ref, bs_ref, out_ref):
    rep = ra_ref[...] + rb_ref[...]
    logits = lax.dot_general(ie_ref[...], rep, (((1,), (1,)), ((), ())),
                             preferred_element_type=jnp.float32)
    bm = bm_ref[...]
    m = jnp.max(bm, axis=0, keepdims=True)
    ssum = jnp.sum(bs_ref[...] * jnp.exp(bm - m), axis=0, keepdims=True)
    out_ref[...] = jnp.exp(logits - m) / ssum


def _softmax_scores(ra, rb, item_emb):
    rrow = pl.BlockSpec((D, D), lambda i: (0, 0))
    iblk = pl.BlockSpec((IBLK, D), lambda i: (i, 0))
    col = pl.BlockSpec((1, 1, D), lambda i: (i, 0, 0))
    stat = pl.BlockSpec((NITEM_BLOCKS, D), lambda i: (0, 0))
    bm, bs = pl.pallas_call(
        _sm1_body,
        grid=(NITEM_BLOCKS,),
        in_specs=[rrow, rrow, iblk],
        out_specs=[col, col],
        out_shape=[jax.ShapeDtypeStruct((NITEM_BLOCKS, 1, D), jnp.float32)] * 2,
        interpret=_I,
    )(ra, rb, item_emb)
    bm = bm.reshape(NITEM_BLOCKS, D)
    bs = bs.reshape(NITEM_BLOCKS, D)
    return pl.pallas_call(
        _sm2_body,
        grid=(NITEM_BLOCKS,),
        in_specs=[rrow, rrow, iblk, stat, stat],
        out_specs=pl.BlockSpec((IBLK, D), lambda i: (i, 0)),
        out_shape=jax.ShapeDtypeStruct((IBLK * NITEM_BLOCKS, D), jnp.float32),
        interpret=_I,
    )(ra, rb, item_emb, bm, bs)


def kernel(session, length, is_core, node_id, edge_index, initial_score,
           keep_rate, item_emb, user_emb, lw_ih, lw_hh, lb_ih, lb_hh,
           mw1, mb1, mw2, W1_w, W2_w, fc_w0, fc_w1):
    n = session.shape[0]
    pad = NPAD - n
    src = edge_index[0]

    sess_p = jnp.pad(session, ((0, pad), (0, 0)))
    len_p = jnp.pad(length, (0, pad), constant_values=1)
    len2 = len_p.reshape(NPAD, 1).astype(jnp.int32)
    nid_p = jnp.pad(node_id, (0, pad))
    core2 = jnp.pad(is_core, (0, pad)).reshape(NPAD, 1).astype(jnp.int32)
    srcm_p = jnp.pad(src.reshape(n, DEG), ((0, pad), (0, 0)))
    init_p = jnp.pad(initial_score.reshape(n, DEG), ((0, pad), (0, 0)))

    sess_flat = sess_p.reshape(-1).astype(jnp.int32)
    nid_p = nid_p.astype(jnp.int32)
    xg = _sc_gather(item_emb, sess_flat, 320, (32, 8)).reshape(NPAD, T * D)
    ug = _sc_gather(user_emb, nid_p, 160, (2, 2))

    ur = _lstm(xg, len2, lw_ih.T, lw_hh.T, lb_ih.reshape(1, 4 * D),
               lb_hh.reshape(1, 4 * D))

    buf = _fuse(ur, ug, core2, W1_w.T)

    idx_perm = srcm_p.T.reshape(-1).astype(jnp.int32)
    idx_row = srcm_p.reshape(-1).astype(jnp.int32)
    srg_f, bg_f, sid_f = _sc_gather_edges(ur, buf, nid_p, idx_perm, idx_row)
    srg = srg_f.reshape(DEG, NPAD, D)
    bg = bg_f.reshape(DEG, NPAD, D)
    sid = sid_f.reshape(NPAD, DEG)

    kr2 = jnp.asarray(keep_rate, jnp.float32).reshape(1, 1)
    agg1, ps, selp = _round1(srg, bg, sid, ur, buf, init_p, kr2,
                             mw1.T, mb1.reshape(1, D), mw2.T)

    ag = _sc_gather(agg1, idx_perm, 320, (24, 8)).reshape(DEG, NPAD, D)
    agg2 = _round2(ag, agg1, ps)

    recent = ur[0:n:CORE_EVERY]
    social = agg2[0:n:CORE_EVERY]
    ncore = n // CORE_EVERY
    rpad = ((0, D - ncore), (0, 0))
    ra = jnp.pad(recent, rpad)
    rb = jnp.pad(social, rpad)
    scorep = _softmax_scores(ra, rb, item_emb)

    score = scorep[:, :ncore].T
    sel = selp[0:n:CORE_EVERY]
    core_user = node_id[0:n:CORE_EVERY]
    return score, sel, core_user


# speedup vs baseline: 1.0817x; 1.0731x over previous
import functools

import jax
import jax.numpy as jnp
from jax import lax
from jax.experimental import pallas as pl
from jax.experimental.pallas import tpu as pltpu
from jax.experimental.pallas import tpu_sc as plsc

_I = False

NPAD = 10240
T = 20
D = 128
DEG = 16
TOPK = 8
CORE_EVERY = 100
BLK = 512
RBLK = 256
IBLK = 2000
NITEM_BLOCKS = 50


def _dot(a, b):
    return jnp.dot(a, b, preferred_element_type=jnp.float32)


NC = 2
NS = 16
NW = NC * NS

_MESH = dict(core_axis_name="c", subcore_axis_name="s")


def _chunks(sub):
    out, o = [], 0
    while o < sub:
        ch = min(128, sub - o)
        out.append((o, ch))
        o += ch
    return out


def _sc_gather(table, idx, sub, u01):
    total = idx.shape[0]
    u0, u1 = u01
    assert NS * (u0 + u1) * sub == total and u0 % 2 == 0 and u1 % 2 == 0
    ck = _chunks(sub)

    @functools.partial(
        pl.kernel, mesh=plsc.VectorSubcoreMesh(**_MESH),
        out_type=jax.ShapeDtypeStruct((total, D), jnp.float32),
        scratch_types=[pltpu.VMEM((sub,), jnp.int32),
                       pltpu.VMEM((sub,), jnp.int32),
                       pltpu.VMEM((sub, D), jnp.float32),
                       pltpu.VMEM((sub, D), jnp.float32),
                       pltpu.SemaphoreType.DMA,
                       pltpu.SemaphoreType.DMA,
                       pltpu.SemaphoreType.DMA])
    def k(table_hbm, idx_hbm, out_hbm, idx0, idx1, rows0, rows1,
          semg, semo0, semo1):
        wid = lax.axis_index("s") * NC + lax.axis_index("c")
        nsub = (u0 + u1) // 2
        base = wid * nsub * sub
        idxb = (idx0, idx1)
        rowsb = (rows0, rows1)
        semo = (semo0, semo1)

        def body(kk, carry):
            hs = [[], []]
            for b in (0, 1):
                off = base + (2 * kk + b) * sub
                @pl.when(kk > 0)
                def _wait():
                    pltpu.make_async_copy(out_hbm.at[pl.ds(0, sub)],
                                          rowsb[b], semo[b]).wait()
                pltpu.sync_copy(idx_hbm.at[pl.ds(off, sub)], idxb[b])
                for (o, ch) in ck:
                    hs[b].append(pltpu.async_copy(
                        table_hbm.at[idxb[b].at[pl.ds(o, ch)]],
                        rowsb[b].at[pl.ds(o, ch)], semg))
            for b in (0, 1):
                off = base + (2 * kk + b) * sub
                for h in hs[b]:
                    h.wait()
                pltpu.async_copy(rowsb[b], out_hbm.at[pl.ds(off, sub)], semo[b])
            return carry

        lax.fori_loop(0, nsub // 2, body, 0)
        for b in (0, 1):
            pltpu.make_async_copy(out_hbm.at[pl.ds(0, sub)], rowsb[b],
                                  semo[b]).wait()

    return k(table, idx)


def _sc_gather_edges(ur, buf, nid, idx_perm, idx_row):
    total = idx_perm.shape[0]
    sub = 160
    u0, u1 = 42, 22
    assert NS * (u0 + u1) * sub == total
    ck = _chunks(sub)

    @functools.partial(
        pl.kernel, mesh=plsc.VectorSubcoreMesh(**_MESH),
        out_type=(jax.ShapeDtypeStruct((total, D), jnp.float32),
                  jax.ShapeDtypeStruct((total, D), jnp.float32),
                  jax.ShapeDtypeStruct((total,), jnp.int32)),
        scratch_types=[pltpu.VMEM((sub,), jnp.int32),
                       pltpu.VMEM((sub,), jnp.int32),
                       pltpu.VMEM((sub,), jnp.int32),
                       pltpu.VMEM((sub,), jnp.int32),
                       pltpu.VMEM((sub, D), jnp.float32),
                       pltpu.VMEM((sub, D), jnp.float32),
                       pltpu.VMEM((sub, D), jnp.float32),
                       pltpu.VMEM((sub, D), jnp.float32),
                       pltpu.VMEM((sub,), jnp.int32),
                       pltpu.VMEM((sub,), jnp.int32),
                       pltpu.SemaphoreType.DMA,
                       pltpu.SemaphoreType.DMA,
                       pltpu.SemaphoreType.DMA])
    def k(ur_hbm, buf_hbm, nid_hbm, idxp_hbm, idxr_hbm,
          srg_hbm, bg_hbm, sid_hbm,
          idxp0, idxp1, idxr0, idxr1, ra0, ra1, rb0, rb1, rs0, rs1,
          semg, semo0, semo1):
        wid = lax.axis_index("s") * NC + lax.axis_index("c")
        nsub = (u0 + u1) // 2
        base = wid * nsub * sub
        idxp = (idxp0, idxp1)
        idxr = (idxr0, idxr1)
        ra = (ra0, ra1)
        rb = (rb0, rb1)
        rs = (rs0, rs1)
        semo = (semo0, semo1)

        def body(kk, carry):
            hs = [[], []]
            for b in (0, 1):
                off = base + (2 * kk + b) * sub

                @pl.when(kk > 0)
                def _wait():
                    pltpu.make_async_copy(srg_hbm.at[pl.ds(0, sub)],
                                          ra[b], semo[b]).wait()
                    pltpu.make_async_copy(bg_hbm.at[pl.ds(0, sub)],
                                          rb[b], semo[b]).wait()
                    pltpu.make_async_copy(sid_hbm.at[pl.ds(0, sub)],
                                          rs[b], semo[b]).wait()
                pltpu.sync_copy(idxp_hbm.at[pl.ds(off, sub)], idxp[b])
                pltpu.sync_copy(idxr_hbm.at[pl.ds(off, sub)], idxr[b])
                for (o, ch) in ck:
                    ip = idxp[b].at[pl.ds(o, ch)]
                    ir = idxr[b].at[pl.ds(o, ch)]
                    hs[b].append(pltpu.async_copy(ur_hbm.at[ip],
                                                  ra[b].at[pl.ds(o, ch)], semg))
                    hs[b].append(pltpu.async_copy(buf_hbm.at[ip],
                                                  rb[b].at[pl.ds(o, ch)], semg))
                    hs[b].append(pltpu.async_copy(nid_hbm.at[ir],
                                                  rs[b].at[pl.ds(o, ch)], semg))
            for b in (0, 1):
                off = base + (2 * kk + b) * sub
                for h in hs[b]:
                    h.wait()
                pltpu.async_copy(ra[b], srg_hbm.at[pl.ds(off, sub)], semo[b])
                pltpu.async_copy(rb[b], bg_hbm.at[pl.ds(off, sub)], semo[b])
                pltpu.async_copy(rs[b], sid_hbm.at[pl.ds(off, sub)], semo[b])
            return carry

        lax.fori_loop(0, nsub // 2, body, 0)
        for b in (0, 1):
            pltpu.make_async_copy(srg_hbm.at[pl.ds(0, sub)], ra[b],
                                  semo[b]).wait()
            pltpu.make_async_copy(bg_hbm.at[pl.ds(0, sub)], rb[b],
                                  semo[b]).wait()
            pltpu.make_async_copy(sid_hbm.at[pl.ds(0, sub)], rs[b],
                                  semo[b]).wait()

    return k(ur, buf, nid, idx_perm, idx_row)


def _lstm_body(x_ref, len_ref, wih_ref, whh_ref, bih_ref, bhh_ref, out_ref):
    x = x_ref[...]
    wih = wih_ref[...]
    whh = whh_ref[...]
    bih = bih_ref[...]
    bhh = bhh_ref[...]
    ln = len_ref[...]
    bsz = x.shape[0]
    h = jnp.zeros((bsz, D), jnp.float32)
    c = jnp.zeros((bsz, D), jnp.float32)
    ho = jnp.zeros((bsz, D), jnp.float32)
    for t in range(T):
        xt = x[:, D * t:D * (t + 1)]
        g = (_dot(xt, wih) + bih + _dot(h, whh)) + bhh
        i = jax.nn.sigmoid(g[:, :D])
        f = jax.nn.sigmoid(g[:, D:2 * D])
        gg = jnp.tanh(g[:, 2 * D:3 * D])
        o = jax.nn.sigmoid(g[:, 3 * D:])
        c = f * c + i * gg
        h = o * jnp.tanh(c)
        ho = jnp.where(ln == (t + 1), h, ho)
    out_ref[...] = ho


def _lstm(xg, len2, wih_t, whh_t, bih, bhh):
    return pl.pallas_call(
        _lstm_body,
        grid=(NPAD // BLK,),
        in_specs=[
            pl.BlockSpec((BLK, T * D), lambda i: (i, 0)),
            pl.BlockSpec((BLK, 1), lambda i: (i, 0)),
            pl.BlockSpec((D, 4 * D), lambda i: (0, 0)),
            pl.BlockSpec((D, 4 * D), lambda i: (0, 0)),
            pl.BlockSpec((1, 4 * D), lambda i: (0, 0)),
            pl.BlockSpec((1, 4 * D), lambda i: (0, 0)),
        ],
        out_specs=pl.BlockSpec((BLK, D), lambda i: (i, 0)),
        out_shape=jax.ShapeDtypeStruct((NPAD, D), jnp.float32),
        interpret=_I,
    )(xg, len2, wih_t, whh_t, bih, bhh)


def _fuse_body(ur_ref, ug_ref, core_ref, w1t_ref, buf_ref):
    ur = ur_ref[...]
    feat = jnp.concatenate([ug_ref[...], ur], axis=1)
    fr = jnp.maximum(_dot(feat, w1t_ref[...]), 0.0)
    buf_ref[...] = jnp.where(core_ref[...] == 1, ur, fr)


def _fuse(ur, ug, core2, w1t):
    row = lambda w: pl.BlockSpec((BLK, w), lambda i: (i, 0))
    return pl.pallas_call(
        _fuse_body,
        grid=(NPAD // BLK,),
        in_specs=[row(D), row(D), row(1),
                  pl.BlockSpec((2 * D, D), lambda i: (0, 0))],
        out_specs=row(D),
        out_shape=jax.ShapeDtypeStruct((NPAD, D), jnp.float32),
        interpret=_I,
    )(ur, ug, core2, w1t)


def _r1_body(srg_ref, bg_ref, sid_ref, ur_ref, buf_ref, init_ref, kr_ref,
             mw1t_ref, mb1_ref, mw2_ref, agg_ref, ps_ref, sel_ref):
    bsz = ur_ref.shape[0]
    ur = ur_ref[...]
    mw1t = mw1t_ref[...]
    mb1 = mb1_ref[...]
    lane = lax.broadcasted_iota(jnp.int32, (bsz, DEG), 1)
    s = jnp.zeros((bsz, DEG), jnp.float32)
    w2row = mw2_ref[...].astype(jnp.bfloat16).astype(jnp.float32)
    for j in range(DEG):
        sr = srg_ref[j]
        feat = jnp.concatenate([sr, sr - ur], axis=1)
        hid = jnp.maximum(_dot(feat, mw1t) + mb1, 0.0)
        hb = hid.astype(jnp.bfloat16).astype(jnp.float32)
        sj = jnp.sum(hb * w2row, axis=1, keepdims=True)
        s = s + jnp.where(lane == j, sj, 0.0)
    lane8 = lax.broadcasted_iota(jnp.int32, (bsz, TOPK), 1)
    work = s
    mask = jnp.zeros((bsz, DEG), jnp.float32)
    selv = jnp.zeros((bsz, TOPK), jnp.int32)
    sid = sid_ref[...]
    for k in range(TOPK):
        m = jnp.max(work, axis=1, keepdims=True)
        am = jnp.min(jnp.where(work == m, lane, DEG), axis=1, keepdims=True)
        oh = lane == am
        mask = mask + jnp.where(oh, 1.0, 0.0)
        selk = jnp.sum(jnp.where(oh, sid, 0), axis=1, keepdims=True)
        selv = selv + jnp.where(lane8 == k, selk, 0)
        work = jnp.where(oh, -3e38, work)
    sel_ref[...] = selv
    kr = kr_ref[0, 0]
    ps = (1.0 - kr) * jax.nn.sigmoid(s - (1.0 - mask) * 1e11) \
        + kr * init_ref[...]
    ps_ref[...] = ps
    coeff = jnp.sum(ps, axis=1, keepdims=True) + 1.0
    fr = jnp.zeros((bsz, D), jnp.float32)
    for j in range(DEG):
        fr = fr + bg_ref[j] * ps[:, j:j + 1]
    agg_ref[...] = (fr + buf_ref[...]) / coeff


def _round1(srg, bg, sid, ur, buf, initm, kr2, mw1t, mb1r, mw2r):
    edge3 = pl.BlockSpec((DEG, RBLK, D), lambda i: (0, i, 0))
    row = lambda w: pl.BlockSpec((RBLK, w), lambda i: (i, 0))
    one = lambda a, b: pl.BlockSpec((a, b), lambda i: (0, 0))
    return pl.pallas_call(
        _r1_body,
        grid=(NPAD // RBLK,),
        in_specs=[edge3, edge3, row(DEG), row(D), row(D), row(DEG),
                  one(1, 1), one(2 * D, D), one(1, D), one(1, D)],
        out_specs=[row(D), row(DEG), row(TOPK)],
        out_shape=[jax.ShapeDtypeStruct((NPAD, D), jnp.float32),
                   jax.ShapeDtypeStruct((NPAD, DEG), jnp.float32),
                   jax.ShapeDtypeStruct((NPAD, TOPK), jnp.int32)],
        interpret=_I,
    )(srg, bg, sid, ur, buf, initm, kr2, mw1t, mb1r, mw2r)


def _r2_body(ag_ref, agg1_ref, ps_ref, out_ref):
    bsz = agg1_ref.shape[0]
    ps = ps_ref[...]
    coeff = jnp.sum(ps, axis=1, keepdims=True) + 1.0
    fr = jnp.zeros((bsz, D), jnp.float32)
    for j in range(DEG):
        fr = fr + ag_ref[j] * ps[:, j:j + 1]
    out_ref[...] = (fr + agg1_ref[...]) / coeff


def _round2(ag, agg1, ps):
    edge3 = pl.BlockSpec((DEG, RBLK, D), lambda i: (0, i, 0))
    row = lambda w: pl.BlockSpec((RBLK, w), lambda i: (i, 0))
    return pl.pallas_call(
        _r2_body,
        grid=(NPAD // RBLK,),
        in_specs=[edge3, row(D), row(DEG)],
        out_specs=row(D),
        out_shape=jax.ShapeDtypeStruct((NPAD, D), jnp.float32),
        interpret=_I,
    )(ag, agg1, ps)


def _sm1_body(ra_ref, rb_ref, ie_ref, bm_ref, bs_ref):
    rep = ra_ref[...] + rb_ref[...]
    logits = lax.dot_general(ie_ref[...], rep, (((1,), (1,)), ((), ())),
                             preferred_element_type=jnp.float32)
    m = jnp.max(logits, axis=0, keepdims=True)
    bm_ref[...] = m[None]
    bs_ref[...] = jnp.sum(jnp.exp(logits - m), axis=0, keepdims=True)[None]


def _sm2_body(ra_ref, rb_ref, ie_ref, bm_ref, bs_ref, out_ref):
    rep = ra_ref[...] + rb_ref[...]
    logits = lax.dot_general(ie_ref[...], rep, (((1,), (1,)), ((), ())),
                             preferred_element_type=jnp.float32)
    bm = bm_ref[...]
    m = jnp.max(bm, axis=0, keepdims=True)
    ssum = jnp.sum(bs_ref[...] * jnp.exp(bm - m), axis=0, keepdims=True)
    out_ref[...] = jnp.exp(logits - m) / ssum


def _softmax_scores(ra, rb, item_emb):
    rrow = pl.BlockSpec((D, D), lambda i: (0, 0))
    iblk = pl.BlockSpec((IBLK, D), lambda i: (i, 0))
    col = pl.BlockSpec((1, 1, D), lambda i: (i, 0, 0))
    stat = pl.BlockSpec((NITEM_BLOCKS, D), lambda i: (0, 0))
    bm, bs = pl.pallas_call(
        _sm1_body,
        grid=(NITEM_BLOCKS,),
        in_specs=[rrow, rrow, iblk],
        out_specs=[col, col],
        out_shape=[jax.ShapeDtypeStruct((NITEM_BLOCKS, 1, D), jnp.float32)] * 2,
        interpret=_I,
    )(ra, rb, item_emb)
    bm = bm.reshape(NITEM_BLOCKS, D)
    bs = bs.reshape(NITEM_BLOCKS, D)
    return pl.pallas_call(
        _sm2_body,
        grid=(NITEM_BLOCKS,),
        in_specs=[rrow, rrow, iblk, stat, stat],
        out_specs=pl.BlockSpec((IBLK, D), lambda i: (i, 0)),
        out_shape=jax.ShapeDtypeStruct((IBLK * NITEM_BLOCKS, D), jnp.float32),
        interpret=_I,
    )(ra, rb, item_emb, bm, bs)


def kernel(session, length, is_core, node_id, edge_index, initial_score,
           keep_rate, item_emb, user_emb, lw_ih, lw_hh, lb_ih, lb_hh,
           mw1, mb1, mw2, W1_w, W2_w, fc_w0, fc_w1):
    n = session.shape[0]
    pad = NPAD - n
    src = edge_index[0]

    sess_p = jnp.pad(session, ((0, pad), (0, 0)))
    len_p = jnp.pad(length, (0, pad), constant_values=1)
    len2 = len_p.reshape(NPAD, 1).astype(jnp.int32)
    nid_p = jnp.pad(node_id, (0, pad))
    core2 = jnp.pad(is_core, (0, pad)).reshape(NPAD, 1).astype(jnp.int32)
    srcm_p = jnp.pad(src.reshape(n, DEG), ((0, pad), (0, 0)))
    init_p = jnp.pad(initial_score.reshape(n, DEG), ((0, pad), (0, 0)))

    sess_flat = sess_p.reshape(-1).astype(jnp.int32)
    nid_p = nid_p.astype(jnp.int32)
    xg = _sc_gather(item_emb, sess_flat, 320, (32, 8)).reshape(NPAD, T * D)
    ug = _sc_gather(user_emb, nid_p, 160, (2, 2))

    ur = _lstm(xg, len2, lw_ih.T, lw_hh.T, lb_ih.reshape(1, 4 * D),
               lb_hh.reshape(1, 4 * D))

    buf = _fuse(ur, ug, core2, W1_w.T)

    idx_perm = srcm_p.T.reshape(-1).astype(jnp.int32)
    idx_row = srcm_p.reshape(-1).astype(jnp.int32)
    srg_f, bg_f, sid_f = _sc_gather_edges(ur, buf, nid_p, idx_perm, idx_row)
    srg = srg_f.reshape(DEG, NPAD, D)
    bg = bg_f.reshape(DEG, NPAD, D)
    sid = sid_f.reshape(NPAD, DEG)

    kr2 = jnp.asarray(keep_rate, jnp.float32).reshape(1, 1)
    agg1, ps, selp = _round1(srg, bg, sid, ur, buf, init_p, kr2,
                             mw1.T, mb1.reshape(1, D), mw2)

    ag = _sc_gather(agg1, idx_perm, 320, (24, 8)).reshape(DEG, NPAD, D)
    agg2 = _round2(ag, agg1, ps)

    recent = ur[0:n:CORE_EVERY]
    social = agg2[0:n:CORE_EVERY]
    ncore = n // CORE_EVERY
    rpad = ((0, D - ncore), (0, 0))
    ra = jnp.pad(recent, rpad)
    rb = jnp.pad(social, rpad)
    scorep = _softmax_scores(ra, rb, item_emb)

    score = scorep[:, :ncore].T
    sel = selp[0:n:CORE_EVERY]
    core_user = node_id[0:n:CORE_EVERY]
    return score, sel, core_user
